# baseline (speedup 1.0000x reference)
import functools

import jax
import jax.numpy as jnp
from jax import lax
from jax.experimental import pallas as pl
from jax.experimental.pallas import tpu as pltpu
from jax.experimental.pallas import tpu_sc as plsc

LANES = 16


def _build(bsz, seq, emb, padded):
    info = plsc.get_sparse_core_info()
    nc, ns = info.num_cores, info.num_subcores
    nw = nc * ns
    bw = bsz // nw
    assert bsz % nw == 0 and bw == 128 and seq % 2 == 0 and emb % 8 == 0
    mesh = plsc.VectorSubcoreMesh(core_axis_name="c", subcore_axis_name="s")

    @functools.partial(
        pl.kernel,
        mesh=mesh,
        out_type=jax.ShapeDtypeStruct((seq, emb, bsz), jnp.float32),
        compiler_params=pltpu.CompilerParams(needs_layout_passes=False),
        scratch_types=[
            pltpu.VMEM((bw, seq), jnp.int32),
            pltpu.VMEM((2, bw), jnp.int32),
            pltpu.VMEM((bw, padded), jnp.float32),
            pltpu.VMEM((bw, padded), jnp.float32),
            pltpu.VMEM((emb, bw), jnp.float32),
            pltpu.VMEM((emb, bw), jnp.float32),
            pltpu.VMEM((emb * (bw + 1),), jnp.float32),
            pltpu.VMEM((emb * (bw + 1),), jnp.float32),
            pltpu.SemaphoreType.DMA,
            pltpu.SemaphoreType.DMA,
        ],
    )
    def k(
        table_hbm, pos_hbm, out_hbm, idx_v, col_v, g0, g1, t0, t1, x0, x1, sem_g, sem_o
    ):
        gbuf = (g0, g1)
        tbuf = (t0, t1)
        xbuf = (x0, x1)
        c = lax.axis_index("c")
        s = lax.axis_index("s")
        wid = s * nc + c
        b0 = wid * bw
        pltpu.sync_copy(pos_hbm.at[pl.ds(b0, bw)], idx_v)

        iota = lax.iota(jnp.int32, LANES)
        j_vecs = [j0 * LANES + iota for j0 in range(bw // LANES)]

        def build_col(t, q):
            tsp = jnp.full((LANES,), 0, jnp.int32) + t
            for j0 in range(bw // LANES):
                col_v[q, pl.ds(j0 * LANES, LANES)] = plsc.load_gather(
                    idx_v, [j_vecs[j0], tsp]
                )

        def gather(t, p, q):
            return pltpu.make_async_copy(
                table_hbm.at[col_v.at[q]], gbuf[p], sem_g
            )

        iota_s = lax.iota(jnp.int32, LANES) * (bw + 1)
        e0_vecs = [iota_s + e0 * (bw + 1) for e0 in range(0, emb, LANES)]

        def transpose(p):
            g, t, x = gbuf[p], tbuf[p], xbuf[p]

            @plsc.parallel_loop(0, bw, unroll=16)
            def _(j):
                jb = jnp.full((LANES,), 0, jnp.int32) + j
                for i, e0 in enumerate(range(0, emb, LANES)):
                    plsc.store_scatter(
                        x, [e0_vecs[i] + jb], g[j, pl.ds(e0, LANES)]
                    )

            @plsc.parallel_loop(0, emb, unroll=16)
            def _(e):
                base = e * (bw + 1)
                for j0 in range(bw // LANES):
                    t[e, pl.ds(j0 * LANES, LANES)] = plsc.load_gather(
                        x, [iota + (base + j0 * LANES)]
                    )

        def out_cp(t, p):
            return pltpu.make_async_copy(
                tbuf[p], out_hbm.at[t, :, pl.ds(b0, bw)], sem_o
            )

        def step(t, p, build_next, drain):
            if build_next:
                build_col(t + 1, (t + 1) % 2)
                gather(t + 1, 1 - p, (t + 1) % 2).start()
            gather(t, p, t % 2).wait()
            if drain:
                out_cp(t - 2, p).wait()
            transpose(p)
            out_cp(t, p).start()

        build_col(0, 0)
        gather(0, 0, 0).start()
        step(0, 0, build_next=True, drain=False)
        step(1, 1, build_next=True, drain=False)

        @pl.loop(2, seq - 2, step=2)
        def _(s0):
            for pp in range(2):
                step(s0 + pp, pp, build_next=True, drain=True)

        step(seq - 2, 0, build_next=True, drain=True)
        step(seq - 1, 1, build_next=False, drain=True)
        out_cp(seq - 2, 0).wait()
        out_cp(seq - 1, 1).wait()

    return k


def kernel(pos, weight):
    b, s = pos.shape
    v, emb = weight.shape
    padded = 128
    k = _build(b, s, emb, padded)
    table = jnp.pad(weight, ((0, 0), (0, padded - emb)))
    res = k(table, pos.astype(jnp.int32))
    return jnp.transpose(res, (2, 0, 1))


# speedup vs baseline: 1.0207x; 1.0207x over previous
import functools

import jax
import jax.numpy as jnp
from jax import lax
from jax.experimental import pallas as pl
from jax.experimental.pallas import tpu as pltpu
from jax.experimental.pallas import tpu_sc as plsc

LANES = 16


def _build(bsz, seq, emb, padded):
    info = plsc.get_sparse_core_info()
    nc, ns = info.num_cores, info.num_subcores
    nw = nc * ns
    bw = bsz // nw
    assert bsz % nw == 0 and bw == 128 and seq % 2 == 0 and emb % 8 == 0
    mesh = plsc.VectorSubcoreMesh(core_axis_name="c", subcore_axis_name="s")

    @functools.partial(
        pl.kernel,
        mesh=mesh,
        out_type=jax.ShapeDtypeStruct((seq, emb, bsz), jnp.float32),
        compiler_params=pltpu.CompilerParams(needs_layout_passes=False),
        scratch_types=[
            pltpu.VMEM((bw, seq), jnp.int32),
            pltpu.VMEM((seq, bw), jnp.int32),
            pltpu.VMEM((bw, padded), jnp.float32),
            pltpu.VMEM((bw, padded), jnp.float32),
            pltpu.VMEM((emb, bw), jnp.float32),
            pltpu.VMEM((emb, bw), jnp.float32),
            pltpu.VMEM((emb * (bw + 1),), jnp.float32),
            pltpu.VMEM((emb * (bw + 1),), jnp.float32),
            pltpu.SemaphoreType.DMA,
            pltpu.SemaphoreType.DMA,
        ],
    )
    def k(
        table_hbm, pos_hbm, out_hbm, idx_v, idx_t, g0, g1, t0, t1, x0, x1, sem_g, sem_o
    ):
        gbuf = (g0, g1)
        tbuf = (t0, t1)
        xbuf = (x0, x1)
        c = lax.axis_index("c")
        s = lax.axis_index("s")
        wid = s * nc + c
        b0 = wid * bw
        pltpu.sync_copy(pos_hbm.at[pl.ds(b0, bw)], idx_v)

        iota = lax.iota(jnp.int32, LANES)
        iota_s = iota * (bw + 1)
        e0_vecs = [iota_s + e0 * (bw + 1) for e0 in range(0, emb, LANES)]

        sblk = 64
        for blk in range((seq + sblk - 1) // sblk):
            s0 = blk * sblk
            hi = min(s0 + sblk, seq)
            starts = [c0 for c0 in range(0, seq, LANES) if s0 <= c0 and c0 + LANES <= min(hi, 128) or (max(s0, 128) <= c0 < hi and c0 + LANES <= seq)]
            if hi == seq and seq % LANES:
                starts.append(seq - LANES)

            @plsc.parallel_loop(0, bw, unroll=8)
            def _(j):
                jb = jnp.full((LANES,), 0, jnp.int32) + j
                for c0 in starts:
                    v = plsc.bitcast(idx_v[j, pl.ds(c0, LANES)], jnp.float32)
                    dest = iota_s + ((c0 - s0) * (bw + 1) + jb)
                    if c0 < s0:
                        plsc.store_scatter(
                            xbuf[0], [dest], v, mask=iota >= (s0 - c0)
                        )
                    else:
                        plsc.store_scatter(xbuf[0], [dest], v)

            @plsc.parallel_loop(0, hi - s0, unroll=8)
            def _(sr):
                base = sr * (bw + 1)
                for j0 in range(bw // LANES):
                    w = plsc.load_gather(xbuf[0], [iota + (base + j0 * LANES)])
                    idx_t[s0 + sr, pl.ds(j0 * LANES, LANES)] = plsc.bitcast(
                        w, jnp.int32
                    )

        def gather(t, p, q):
            return pltpu.make_async_copy(
                table_hbm.at[idx_t.at[t]], gbuf[p], sem_g
            )

        def transpose(p):
            g, t, x = gbuf[p], tbuf[p], xbuf[p]

            @plsc.parallel_loop(0, bw, unroll=8)
            def _(j):
                jb = jnp.full((LANES,), 0, jnp.int32) + j
                for i, e0 in enumerate(range(0, emb, LANES)):
                    plsc.store_scatter(
                        x, [e0_vecs[i] + jb], g[j, pl.ds(e0, LANES)]
                    )

            @plsc.parallel_loop(0, emb, unroll=8)
            def _(e):
                base = e * (bw + 1)
                for j0 in range(bw // LANES):
                    t[e, pl.ds(j0 * LANES, LANES)] = plsc.load_gather(
                        x, [iota + (base + j0 * LANES)]
                    )

        def out_cp(t, p):
            return pltpu.make_async_copy(
                tbuf[p], out_hbm.at[t, :, pl.ds(b0, bw)], sem_o
            )

        def step(t, p, build_next, drain):
            if build_next:
                gather(t + 1, 1 - p, (t + 1) % 2).start()
            gather(t, p, t % 2).wait()
            if drain:
                out_cp(t - 2, p).wait()
            transpose(p)
            out_cp(t, p).start()

        gather(0, 0, 0).start()
        step(0, 0, build_next=True, drain=False)
        step(1, 1, build_next=True, drain=False)

        @pl.loop(2, seq - 2, step=2)
        def _(s0):
            for pp in range(2):
                step(s0 + pp, pp, build_next=True, drain=True)

        step(seq - 2, 0, build_next=True, drain=True)
        step(seq - 1, 1, build_next=False, drain=True)
        out_cp(seq - 2, 0).wait()
        out_cp(seq - 1, 1).wait()

    return k


def kernel(pos, weight):
    b, s = pos.shape
    v, emb = weight.shape
    padded = 128
    k = _build(b, s, emb, padded)
    table = jnp.pad(weight, ((0, 0), (0, padded - emb)))
    res = k(table, pos.astype(jnp.int32))
    return jnp.transpose(res, (2, 0, 1))
